# baseline (speedup 1.0000x reference)
import functools

import jax
import jax.numpy as jnp
from jax import lax
from jax.experimental import pallas as pl
from jax.experimental.pallas import tpu as pltpu


def _extrema_kernel(x_hbm, o_hbm, xbuf, zbuf, acc_ref, zsems, ssems, rsem, *,
                    pool_size: int, width: int, zrows: int, n_chunks: int):
    n, c_dim, pw = xbuf.shape
    row = o_hbm.shape[1]

    pltpu.make_async_copy(x_hbm.at[:, :, pl.ds(0, pw)], xbuf, rsem).start()

    zbuf[...] = jnp.zeros(zbuf.shape, zbuf.dtype)
    for k in range(n_chunks):
        pltpu.make_async_copy(
            zbuf, o_hbm.at[pl.ds(k * zrows, zrows), :], zsems.at[k]).start()

    pltpu.make_async_copy(x_hbm.at[:, :, pl.ds(0, pw)], xbuf, rsem).wait()
    xw = xbuf[...]
    lane = lax.broadcasted_iota(jnp.int32, xw.shape, 2)
    inwin = lane % width < pool_size
    aw = jnp.where(inwin, jnp.abs(xw), -1.0)
    m = jnp.max(aw, axis=-1, keepdims=True)
    cand = jnp.where(aw == m, lane, jnp.int32(pw))
    idx = jnp.min(cand, axis=-1, keepdims=True)
    col = lax.broadcasted_iota(jnp.int32, (1, pw), 1)
    hit = col == idx[:, 0, :]
    for c in range(1, c_dim):
        hit = hit | (col == idx[:, c, :])
    acc_ref[...] = jnp.where(hit, xw[:, 0, :], 0.0).astype(acc_ref.dtype)

    for k in range(n_chunks):
        pltpu.make_async_copy(
            zbuf, o_hbm.at[pl.ds(k * zrows, zrows), :], zsems.at[k]).wait()
        pltpu.make_async_copy(
            acc_ref.at[pl.ds(k * zrows, zrows), :],
            o_hbm.at[pl.ds(k * zrows, zrows), pl.ds(0, pw)],
            ssems.at[k],
        ).start()
    for k in range(n_chunks):
        pltpu.make_async_copy(
            acc_ref.at[pl.ds(k * zrows, zrows), :],
            o_hbm.at[pl.ds(k * zrows, zrows), pl.ds(0, pw)],
            ssems.at[k],
        ).wait()


def _extrema_pool_indices_2d(x, pool_size: int):
    N, C, H, W = x.shape
    HW = H * W
    row = C * HW
    x3 = x.reshape(N, C, HW)
    pw = pool_size * W

    zrows = min(256, N)
    n_chunks = N // zrows

    out2 = pl.pallas_call(
        functools.partial(_extrema_kernel, pool_size=pool_size, width=W,
                          zrows=zrows, n_chunks=n_chunks),
        out_shape=jax.ShapeDtypeStruct((N, row), x.dtype),
        in_specs=[pl.BlockSpec(memory_space=pl.ANY)],
        out_specs=pl.BlockSpec(memory_space=pl.ANY),
        scratch_shapes=[
            pltpu.VMEM((N, C, pw), x.dtype),
            pltpu.VMEM((zrows, row), x.dtype),
            pltpu.VMEM((N, pw), x.dtype),
            pltpu.SemaphoreType.DMA((n_chunks,)),
            pltpu.SemaphoreType.DMA((n_chunks,)),
            pltpu.SemaphoreType.DMA,
        ],
        compiler_params=pltpu.CompilerParams(
            vmem_limit_bytes=64 * 1024 * 1024,
        ),
        cost_estimate=pl.CostEstimate(
            flops=10 * N * C * pw,
            transcendentals=0,
            bytes_accessed=(N * row + N * C * pw) * x.dtype.itemsize,
        ),
    )(x3)
    return out2.reshape(N, C, H, W)


def kernel(x):
    return _extrema_pool_indices_2d(x, 4)
